# baseline (speedup 1.0000x reference)
import functools

import jax
import jax.numpy as jnp
from jax.experimental import pallas as pl
from jax.experimental.pallas import tpu as pltpu

B, N, M = 4, 2048, 1024
K0, K1, KA = 32, 32, 16
NT = 256


def _lrelu(v):
    return jnp.where(v > 0, v, 0.2 * v)


def _head_pool_body(xt_ref, w_ref, ymax_ref, ysum_ref):
    nt = pl.program_id(1)
    y = _lrelu(
        jax.lax.dot_general(
            xt_ref[0], w_ref[...], (((1,), (1,)), ((), ())),
            preferred_element_type=jnp.float32,
        )
    )
    ymax = jnp.max(y, axis=0, keepdims=True)
    ysum = jnp.sum(y, axis=0, keepdims=True)

    @pl.when(nt == 0)
    def _():
        ymax_ref[0] = ymax
        ysum_ref[0] = ysum

    @pl.when(nt != 0)
    def _():
        ymax_ref[0] = jnp.maximum(ymax_ref[0], ymax)
        ysum_ref[0] = ysum_ref[0] + ysum


def _head_pool(x_tmp_t, conv_W):
    grid = (B, N // NT)
    ymax, ysum = pl.pallas_call(
        _head_pool_body,
        grid=grid,
        in_specs=[
            pl.BlockSpec((1, NT, 128), lambda b, n: (b, n, 0)),
            pl.BlockSpec((1024, 128), lambda b, n: (0, 0)),
        ],
        out_specs=[
            pl.BlockSpec((1, 1, 1024), lambda b, n: (b, 0, 0)),
            pl.BlockSpec((1, 1, 1024), lambda b, n: (b, 0, 0)),
        ],
        out_shape=[
            jax.ShapeDtypeStruct((B, 1, 1024), jnp.float32),
            jax.ShapeDtypeStruct((B, 1, 1024), jnp.float32),
        ],
    )(x_tmp_t, conv_W)
    return ymax[:, 0], ysum[:, 0]


def _head_mlp_body(xt_ref, g_ref, w2x_ref, w2g_ref, w3_ref, w4_ref, out_ref,
                   gc_ref):
    nt = pl.program_id(1)

    @pl.when(nt == 0)
    def _():
        gc_ref[...] = jax.lax.dot_general(
            g_ref[0], w2g_ref[...], (((1,), (1,)), ((), ())),
            preferred_element_type=jnp.float32,
        )

    x = xt_ref[0]
    t1 = _lrelu(
        jax.lax.dot_general(
            x, w2x_ref[...], (((1,), (1,)), ((), ())),
            preferred_element_type=jnp.float32,
        )
        + gc_ref[...]
    )
    t2 = _lrelu(
        jax.lax.dot_general(
            t1, w3_ref[...], (((1,), (1,)), ((), ())),
            preferred_element_type=jnp.float32,
        )
    )
    out_ref[0] = jax.lax.dot_general(
        t2, w4_ref[...], (((1,), (1,)), ((), ())),
        preferred_element_type=jnp.float32,
    )


def _head_mlp(x_tmp_t, g, w2, w3, w4):
    w2g = w2[:, :2112]
    w2x = w2[:, 2112:]
    w4p = jnp.zeros((64, 256), jnp.float32).at[:50].set(w4)
    grid = (B, N // NT)
    out = pl.pallas_call(
        _head_mlp_body,
        grid=grid,
        in_specs=[
            pl.BlockSpec((1, NT, 128), lambda b, n: (b, n, 0)),
            pl.BlockSpec((1, 1, 2112), lambda b, n: (b, 0, 0)),
            pl.BlockSpec((1024, 128), lambda b, n: (0, 0)),
            pl.BlockSpec((1024, 2112), lambda b, n: (0, 0)),
            pl.BlockSpec((256, 1024), lambda b, n: (0, 0)),
            pl.BlockSpec((64, 256), lambda b, n: (0, 0)),
        ],
        out_specs=pl.BlockSpec((1, NT, 64), lambda b, n: (b, n, 0)),
        out_shape=jax.ShapeDtypeStruct((B, N, 64), jnp.float32),
        scratch_shapes=[pltpu.VMEM((1, 1024), jnp.float32)],
    )(x_tmp_t, g[:, None, :], w2x, w2g, w3, w4p)
    return out


def _knn_idx(x, k):
    xt = jnp.transpose(x, (0, 2, 1))
    inner = -2.0 * jnp.einsum('bnc,bmc->bnm', xt, xt)
    sq = jnp.sum(xt * xt, axis=-1)
    neg = -(sq[:, :, None] + inner + sq[:, None, :])
    _, idx = jax.lax.top_k(neg, k)
    return idx


def _gather_nbrs(x, idx):
    b, c, n = x.shape
    k = idx.shape[-1]
    xt = jnp.transpose(x, (0, 2, 1))
    g = jnp.take_along_axis(xt, idx.reshape(b, n * k)[:, :, None], axis=1)
    return jnp.transpose(g.reshape(b, n, k, c), (0, 3, 1, 2))


def _conv1d(W, v):
    return jnp.einsum('oc,bcn->bon', W, v)


def _edge_conv(x, W, k):
    idx = _knn_idx(x, k)
    nbr = _gather_nbrs(x, idx)
    ctr = x[:, :, :, None]
    feat = jnp.concatenate([nbr - ctr, jnp.broadcast_to(ctr, nbr.shape)], axis=1)
    y = _lrelu(jnp.einsum('oc,bcnk->bonk', W, feat))
    return jnp.max(y, axis=-1)


def _n2p_attention(x, Wq, Wk, Wv, Wf, k):
    idx = _knn_idx(x, k)
    q = _conv1d(Wq, x)
    kk = _conv1d(Wk, x)
    vv = _conv1d(Wv, x)
    kn = _gather_nbrs(kk, idx)
    vn = _gather_nbrs(vv, idx)
    att = jax.nn.softmax(
        jnp.einsum('bdn,bdnk->bnk', q, kn) / jnp.sqrt(1.0 * q.shape[1]), axis=-1)
    out = x + jnp.einsum('bnk,bdnk->bdn', att, vn)
    return out + _lrelu(_conv1d(Wf, out))


def _gather_by_idx(x, idx):
    xt = jnp.transpose(x, (0, 2, 1))
    return jnp.transpose(jnp.take_along_axis(xt, idx[:, :, None], axis=1), (0, 2, 1))


def _downsample_global(x, w, m):
    s = jnp.einsum('c,bcn->bn', w, x)
    _, isel = jax.lax.top_k(s, m)
    return _gather_by_idx(x, isel), isel


def _upsample_interp(x_skip, x_coarse, xyz_c, xyz_f, Wu):
    ft = jnp.transpose(xyz_f, (0, 2, 1))
    ct = jnp.transpose(xyz_c, (0, 2, 1))
    d2 = (jnp.sum(ft * ft, -1)[:, :, None]
          - 2.0 * jnp.einsum('bnc,bmc->bnm', ft, ct)
          + jnp.sum(ct * ct, -1)[:, None, :])
    negd, idx3 = jax.lax.top_k(-d2, 3)
    w = 1.0 / (jnp.maximum(-negd, 0.0) + 1e-8)
    w = w / jnp.sum(w, axis=-1, keepdims=True)
    xct = jnp.transpose(x_coarse, (0, 2, 1))
    b, n, _ = idx3.shape
    g = jnp.take_along_axis(
        xct, idx3.reshape(b, n * 3)[:, :, None], axis=1).reshape(b, n, 3, xct.shape[-1])
    interp = jnp.einsum('bnk,bnkc->bcn', w, g)
    return _lrelu(_conv1d(Wu, jnp.concatenate([x_skip, interp], axis=1)))


def kernel(x, category_id, emb0_W, emb1_W, att0_Wq, att0_Wk, att0_Wv, att0_Wf,
           att1_Wq, att1_Wk, att1_Wv, att1_Wf, att2_Wq, att2_Wk, att2_Wv,
           att2_Wf, ds_w, us_W, conv_W, conv1_W, conv2_W, conv3_W, conv4_W):
    x_xyz = x[:, :3, :]
    x0 = _edge_conv(x, emb0_W, K0)
    x1 = _edge_conv(x0, emb1_W, K1)
    xf = jnp.concatenate([x0, x1], axis=1)
    xf = _n2p_attention(xf, att0_Wq, att0_Wk, att0_Wv, att0_Wf, KA)
    xd, isel = _downsample_global(xf, ds_w, M)
    xd = _n2p_attention(xd, att1_Wq, att1_Wk, att1_Wv, att1_Wf, KA)
    xyz_d = _gather_by_idx(x_xyz, isel)
    xu = _upsample_interp(xf, xd, xyz_d, x_xyz, us_W)
    x_tmp = _n2p_attention(xu, att2_Wq, att2_Wk, att2_Wv, att2_Wf, KA)

    x_tmp_t = jnp.transpose(x_tmp, (0, 2, 1))
    ymax, ysum = _head_pool(x_tmp_t, conv_W)
    yavg = ysum / N
    cid = _lrelu(jnp.einsum('oc,bcx->box', conv1_W, category_id))[:, :, 0]
    g = jnp.concatenate([ymax, yavg, cid], axis=1)
    out = _head_mlp(x_tmp_t, g, conv2_W, conv3_W, conv4_W)
    return jnp.transpose(out[:, :, :50], (0, 2, 1))


# speedup vs baseline: 3.8507x; 3.8507x over previous
import functools

import jax
import jax.numpy as jnp
from jax.experimental import pallas as pl
from jax.experimental.pallas import tpu as pltpu

B, N, M = 4, 2048, 1024
K0, K1, KA = 32, 32, 16
NT = 256


def _lrelu(v):
    return jnp.where(v > 0, v, 0.2 * v)


def _head_pool_body(xt_ref, w_ref, ymax_ref, ysum_ref):
    nt = pl.program_id(1)
    y = _lrelu(
        jax.lax.dot_general(
            xt_ref[0], w_ref[...], (((1,), (1,)), ((), ())),
            preferred_element_type=jnp.float32,
        )
    )
    ymax = jnp.max(y, axis=0, keepdims=True)
    ysum = jnp.sum(y, axis=0, keepdims=True)

    @pl.when(nt == 0)
    def _():
        ymax_ref[0] = ymax
        ysum_ref[0] = ysum

    @pl.when(nt != 0)
    def _():
        ymax_ref[0] = jnp.maximum(ymax_ref[0], ymax)
        ysum_ref[0] = ysum_ref[0] + ysum


def _head_pool(x_tmp_t, conv_W):
    grid = (B, N // NT)
    ymax, ysum = pl.pallas_call(
        _head_pool_body,
        grid=grid,
        in_specs=[
            pl.BlockSpec((1, NT, 128), lambda b, n: (b, n, 0)),
            pl.BlockSpec((1024, 128), lambda b, n: (0, 0)),
        ],
        out_specs=[
            pl.BlockSpec((1, 1, 1024), lambda b, n: (b, 0, 0)),
            pl.BlockSpec((1, 1, 1024), lambda b, n: (b, 0, 0)),
        ],
        out_shape=[
            jax.ShapeDtypeStruct((B, 1, 1024), jnp.float32),
            jax.ShapeDtypeStruct((B, 1, 1024), jnp.float32),
        ],
    )(x_tmp_t, conv_W)
    return ymax[:, 0], ysum[:, 0]


def _head_mlp_body(xt_ref, g_ref, w2x_ref, w2g_ref, w3_ref, w4_ref, out_ref,
                   gc_ref):
    nt = pl.program_id(1)

    @pl.when(nt == 0)
    def _():
        gc_ref[...] = jax.lax.dot_general(
            g_ref[0], w2g_ref[...], (((1,), (1,)), ((), ())),
            preferred_element_type=jnp.float32,
        )

    x = xt_ref[0]
    t1 = _lrelu(
        jax.lax.dot_general(
            x, w2x_ref[...], (((1,), (1,)), ((), ())),
            preferred_element_type=jnp.float32,
        )
        + gc_ref[...]
    )
    t2 = _lrelu(
        jax.lax.dot_general(
            t1, w3_ref[...], (((1,), (1,)), ((), ())),
            preferred_element_type=jnp.float32,
        )
    )
    out_ref[0] = jax.lax.dot_general(
        t2, w4_ref[...], (((1,), (1,)), ((), ())),
        preferred_element_type=jnp.float32,
    )


def _head_mlp(x_tmp_t, g, w2, w3, w4):
    w2g = w2[:, :2112]
    w2x = w2[:, 2112:]
    w4p = jnp.zeros((64, 256), jnp.float32).at[:50].set(w4)
    grid = (B, N // NT)
    out = pl.pallas_call(
        _head_mlp_body,
        grid=grid,
        in_specs=[
            pl.BlockSpec((1, NT, 128), lambda b, n: (b, n, 0)),
            pl.BlockSpec((1, 1, 2112), lambda b, n: (b, 0, 0)),
            pl.BlockSpec((1024, 128), lambda b, n: (0, 0)),
            pl.BlockSpec((1024, 2112), lambda b, n: (0, 0)),
            pl.BlockSpec((256, 1024), lambda b, n: (0, 0)),
            pl.BlockSpec((64, 256), lambda b, n: (0, 0)),
        ],
        out_specs=pl.BlockSpec((1, NT, 64), lambda b, n: (b, n, 0)),
        out_shape=jax.ShapeDtypeStruct((B, N, 64), jnp.float32),
        scratch_shapes=[pltpu.VMEM((1, 1024), jnp.float32)],
    )(x_tmp_t, g[:, None, :], w2x, w2g, w3, w4p)
    return out


def _dg(a, b, ca, cb):
    return jax.lax.dot_general(
        a, b, (((ca,), (cb,)), ((), ())),
        precision=jax.lax.Precision.HIGHEST,
        preferred_element_type=jnp.float32)


def _dgd(a, b, ca, cb):
    return jax.lax.dot_general(
        a, b, (((ca,), (cb,)), ((), ())),
        preferred_element_type=jnp.float32)


NEG_INF = float('-inf')


def _edge_conv_body(k, c, xt_ref, xf_ref, w_ref, out_ref, s_ref, mx_ref):
    x_full = xf_ref[0]
    x_t = xt_ref[0]
    cp = x_t.shape[1]
    n = x_full.shape[0]

    sq_j = jnp.sum(x_full * x_full, axis=1, keepdims=True)
    sq_i = _dg(jnp.ones((1, cp), jnp.float32), x_t * x_t, 1, 1)
    inner = -2.0 * _dgd(x_full, x_t, 1, 1)
    s_ref[...] = -((sq_i + inner) + sq_j)

    mx_ref[...] = jnp.full(mx_ref.shape, NEG_INF, jnp.float32)
    iota = jax.lax.broadcasted_iota(jnp.int32, (n, 1), 0)

    def body(_, carry):
        s = s_ref[...]
        m = jnp.max(s, axis=0, keepdims=True)
        am = jnp.min(jnp.where(s == m, iota, n), axis=0, keepdims=True)
        e = iota == am
        nbr = _dg(e.astype(jnp.float32), x_full, 0, 0)
        feat = jnp.concatenate(
            [(nbr - x_t)[:, :c], x_t[:, :c]], axis=1)
        y = _dgd(feat, w_ref[...], 1, 1)
        mx_ref[...] = jnp.maximum(mx_ref[...], y)
        s_ref[...] = jnp.where(e, NEG_INF, s)
        return carry

    jax.lax.fori_loop(0, k, body, 0)
    out_ref[0] = _lrelu(mx_ref[...])


def _edge_conv(xt, W, k, r=256):
    b, np_, cp = xt.shape
    o = W.shape[0]
    c = W.shape[1] // 2
    out = pl.pallas_call(
        functools.partial(_edge_conv_body, k, c),
        grid=(b, np_ // r),
        in_specs=[
            pl.BlockSpec((1, r, cp), lambda bb, n: (bb, n, 0)),
            pl.BlockSpec((1, np_, cp), lambda bb, n: (bb, 0, 0)),
            pl.BlockSpec((o, 2 * c), lambda bb, n: (0, 0)),
        ],
        out_specs=pl.BlockSpec((1, r, o), lambda bb, n: (bb, n, 0)),
        out_shape=jax.ShapeDtypeStruct((b, np_, o), jnp.float32),
        scratch_shapes=[
            pltpu.VMEM((np_, r), jnp.float32),
            pltpu.VMEM((r, o), jnp.float32),
        ],
    )(xt, xt, W)
    return out


def _att_body(k, xt_ref, xf_ref, wq_ref, wk_ref, wv_ref, wf_ref, out_ref,
              s_ref, mk_ref, kf_ref, vf_ref):
    nt = pl.program_id(1)
    x_full = xf_ref[0]
    x_t = xt_ref[0]

    @pl.when(nt == 0)
    def _():
        kf_ref[...] = _dg(x_full, wk_ref[...], 1, 1)
        vf_ref[...] = _dg(x_full, wv_ref[...], 1, 1)

    n = x_full.shape[0]
    sq_j = jnp.sum(x_full * x_full, axis=1, keepdims=True)
    sq_i = _dg(jnp.ones((1, 128), jnp.float32), x_t * x_t, 1, 1)
    inner = -2.0 * _dgd(x_full, x_t, 1, 1)
    s_ref[...] = -((sq_i + inner) + sq_j)
    mk_ref[...] = jnp.zeros(mk_ref.shape, jnp.float32)
    iota = jax.lax.broadcasted_iota(jnp.int32, (n, 1), 0)

    def body(_, carry):
        s = s_ref[...]
        m = jnp.max(s, axis=0, keepdims=True)
        am = jnp.min(jnp.where(s == m, iota, n), axis=0, keepdims=True)
        e = iota == am
        mk_ref[...] = jnp.where(e, 1.0, mk_ref[...])
        s_ref[...] = jnp.where(e, NEG_INF, s)
        return carry

    jax.lax.fori_loop(0, k, body, 0)

    q_t = _dg(x_t, wq_ref[...], 1, 1)
    scores = _dg(kf_ref[...], q_t, 1, 1) / jnp.sqrt(128.0)
    mk = mk_ref[...] > 0.0
    sm = jnp.max(jnp.where(mk, scores, NEG_INF), axis=0, keepdims=True)
    p = jnp.where(mk, jnp.exp(scores - sm), 0.0)
    att = p / jnp.sum(p, axis=0, keepdims=True)
    o = _dg(att, vf_ref[...], 0, 0)
    out1 = x_t + o
    out_ref[0] = out1 + _lrelu(_dg(out1, wf_ref[...], 1, 1))


def _attention(xt, wq, wk, wv, wf, k, r=256):
    b, np_, d = xt.shape
    out = pl.pallas_call(
        functools.partial(_att_body, k),
        grid=(b, np_ // r),
        in_specs=[
            pl.BlockSpec((1, r, d), lambda bb, n: (bb, n, 0)),
            pl.BlockSpec((1, np_, d), lambda bb, n: (bb, 0, 0)),
            pl.BlockSpec((d, d), lambda bb, n: (0, 0)),
            pl.BlockSpec((d, d), lambda bb, n: (0, 0)),
            pl.BlockSpec((d, d), lambda bb, n: (0, 0)),
            pl.BlockSpec((d, d), lambda bb, n: (0, 0)),
        ],
        out_specs=pl.BlockSpec((1, r, d), lambda bb, n: (bb, n, 0)),
        out_shape=jax.ShapeDtypeStruct((b, np_, d), jnp.float32),
        scratch_shapes=[
            pltpu.VMEM((np_, r), jnp.float32),
            pltpu.VMEM((np_, r), jnp.float32),
            pltpu.VMEM((np_, d), jnp.float32),
            pltpu.VMEM((np_, d), jnp.float32),
        ],
    )(xt, xt, wq, wk, wv, wf)
    return out


def _shift_cumsum(col, n):
    sh = 1
    while sh < n:
        shifted = jnp.concatenate(
            [jnp.zeros((sh, 1), jnp.float32), col[:-sh]], axis=0)
        col = col + shifted
        sh *= 2
    return col


def _ds_body(m_sel, xf_ref, xyz_ref, w_ref, xd_ref, xyzd_ref):
    xf = xf_ref[0]
    xyz = xyz_ref[0]
    n = xf.shape[0]
    s = _dgd(xf, w_ref[...], 1, 1)

    bits = jax.lax.bitcast_convert_type(s, jnp.int32)
    key = jnp.where(bits >= 0, bits, bits ^ jnp.int32(0x7FFFFFFF))

    def bis(_, lohi):
        lo, hi = lohi
        mid = (lo >> 1) + (hi >> 1) + (lo & hi & 1)
        cnt = jnp.sum((key > mid).astype(jnp.int32))
        return jnp.where(cnt <= m_sel - 1, lo, mid), jnp.where(
            cnt <= m_sel - 1, mid, hi)

    lo0 = jnp.int32(-2**31)
    hi0 = jnp.int32(2**31 - 1)
    lo, hi = jax.lax.fori_loop(0, 32, bis, (lo0, hi0))
    vstar = hi

    gt = key > vstar
    eq = key == vstar
    c1 = jnp.sum(gt.astype(jnp.float32))
    cs_eq = _shift_cumsum(eq.astype(jnp.float32), n)
    mask = gt | (eq & (cs_eq <= (jnp.float32(m_sel) - c1)))
    maskf = mask.astype(jnp.float32)
    pos = (_shift_cumsum(maskf, n) - 1.0).astype(jnp.int32)
    lane = jax.lax.broadcasted_iota(jnp.int32, (1, m_sel), 1)
    pm = jnp.where(mask, (pos == lane).astype(jnp.float32), 0.0)
    xd_ref[0] = _dg(pm, xf, 0, 0)
    xyzd_ref[0] = _dg(pm, xyz, 0, 0)


def _downsample(xf, xyz, ds_w, m_sel):
    b, n, d = xf.shape
    xd, xyzd = pl.pallas_call(
        functools.partial(_ds_body, m_sel),
        grid=(b,),
        in_specs=[
            pl.BlockSpec((1, n, d), lambda bb: (bb, 0, 0)),
            pl.BlockSpec((1, n, 8), lambda bb: (bb, 0, 0)),
            pl.BlockSpec((1, d), lambda bb: (0, 0)),
        ],
        out_specs=[
            pl.BlockSpec((1, m_sel, d), lambda bb: (bb, 0, 0)),
            pl.BlockSpec((1, m_sel, 8), lambda bb: (bb, 0, 0)),
        ],
        out_shape=[
            jax.ShapeDtypeStruct((b, m_sel, d), jnp.float32),
            jax.ShapeDtypeStruct((b, m_sel, 8), jnp.float32),
        ],
    )(xf, xyz, ds_w.reshape(1, d))
    return xd, xyzd


def _us_body(xyzt_ref, xyzd_ref, xd_ref, skip_ref, wu_ref, out_ref):
    xyz_t = xyzt_ref[0]
    xyz_d = xyzd_ref[0]
    xd = xd_ref[0]
    skip = skip_ref[0]

    m_sel = xyz_d.shape[0]
    sq_j = jnp.sum(xyz_d * xyz_d, axis=1, keepdims=True)
    sq_i = _dg(jnp.ones((1, 8), jnp.float32), xyz_t * xyz_t, 1, 1)
    d2 = (sq_i - 2.0 * _dgd(xyz_d, xyz_t, 1, 1)) + sq_j
    nd = -d2

    r = xyz_t.shape[0]
    acc = jnp.zeros((128, r), jnp.float32)
    wsum = jnp.zeros((1, r), jnp.float32)
    iota = jax.lax.broadcasted_iota(jnp.int32, (m_sel, 1), 0)
    for _ in range(3):
        m = jnp.max(nd, axis=0, keepdims=True)
        am = jnp.min(jnp.where(nd == m, iota, m_sel), axis=0, keepdims=True)
        e = iota == am
        w = 1.0 / (jnp.maximum(-m, 0.0) + 1e-8)
        acc = acc + w * _dg(xd, e.astype(jnp.float32), 0, 0)
        wsum = wsum + w
        nd = jnp.where(e, NEG_INF, nd)
    interp_t = acc / wsum

    wu = wu_ref[...]
    out_ref[0] = _lrelu(_dg(skip, wu[:, :128], 1, 1)
                        + _dg(interp_t, wu[:, 128:], 0, 1))


def _upsample(xf_skip, xd, xyzd, xyzt, us_W, r=256):
    b, n, d = xf_skip.shape
    m_sel = xd.shape[1]
    out = pl.pallas_call(
        _us_body,
        grid=(b, n // r),
        in_specs=[
            pl.BlockSpec((1, r, 8), lambda bb, nn: (bb, nn, 0)),
            pl.BlockSpec((1, m_sel, 8), lambda bb, nn: (bb, 0, 0)),
            pl.BlockSpec((1, m_sel, d), lambda bb, nn: (bb, 0, 0)),
            pl.BlockSpec((1, r, d), lambda bb, nn: (bb, nn, 0)),
            pl.BlockSpec((d, 2 * d), lambda bb, nn: (0, 0)),
        ],
        out_specs=pl.BlockSpec((1, r, d), lambda bb, nn: (bb, nn, 0)),
        out_shape=jax.ShapeDtypeStruct((b, n, d), jnp.float32),
    )(xyzt, xyzd, xd, xf_skip, us_W)
    return out


def _j_knn(xt, k):
    inner = -2.0 * jnp.einsum('bnc,bmc->bnm', xt, xt)
    sq = jnp.sum(xt * xt, axis=-1)
    neg = -(sq[:, :, None] + inner + sq[:, None, :])
    _, idx = jax.lax.top_k(neg, k)
    return idx


def _j_gather(xt, idx):
    b, n, c = xt.shape
    k = idx.shape[-1]
    g = jnp.take_along_axis(xt, idx.reshape(b, n * k)[:, :, None], axis=1)
    return g.reshape(b, n, k, c)


def _j_edge_conv(xt, W, k):
    idx = _j_knn(xt, k)
    nbr = _j_gather(xt, idx)
    ctr = xt[:, :, None, :]
    feat = jnp.concatenate([nbr - ctr, jnp.broadcast_to(ctr, nbr.shape)], -1)
    y = _lrelu(jnp.einsum('oc,bnkc->bnko', W, feat))
    return jnp.max(y, axis=2)


def _j_attention(xt, Wq, Wk, Wv, Wf, k):
    idx = _j_knn(xt, k)
    q = jnp.einsum('oc,bnc->bno', Wq, xt)
    kk = jnp.einsum('oc,bnc->bno', Wk, xt)
    vv = jnp.einsum('oc,bnc->bno', Wv, xt)
    kn = _j_gather(kk, idx)
    vn = _j_gather(vv, idx)
    att = jax.nn.softmax(
        jnp.einsum('bnd,bnkd->bnk', q, kn) / jnp.sqrt(1.0 * q.shape[2]), -1)
    out = xt + jnp.einsum('bnk,bnkd->bnd', att, vn)
    return out + _lrelu(jnp.einsum('oc,bnc->bno', Wf, out))


def _j_downsample(xt, xyz8, w, m):
    s = jnp.einsum('c,bnc->bn', w, xt)
    _, isel = jax.lax.top_k(s, m)
    return (jnp.take_along_axis(xt, isel[:, :, None], axis=1),
            jnp.take_along_axis(xyz8, isel[:, :, None], axis=1))


def _j_upsample(skip, xd, xyzd, xyzf, Wu):
    ft = xyzf[:, :, :3]
    ct = xyzd[:, :, :3]
    d2 = (jnp.sum(ft * ft, -1)[:, :, None]
          - 2.0 * jnp.einsum('bnc,bmc->bnm', ft, ct)
          + jnp.sum(ct * ct, -1)[:, None, :])
    negd, idx3 = jax.lax.top_k(-d2, 3)
    w = 1.0 / (jnp.maximum(-negd, 0.0) + 1e-8)
    w = w / jnp.sum(w, axis=-1, keepdims=True)
    b, n, _ = idx3.shape
    g = jnp.take_along_axis(xd, idx3.reshape(b, n * 3)[:, :, None],
                            axis=1).reshape(b, n, 3, xd.shape[-1])
    interp = jnp.einsum('bnk,bnkc->bnc', w, g)
    feat = jnp.concatenate([skip, interp], axis=-1)
    return _lrelu(jnp.einsum('oc,bnc->bno', Wu, feat))


def kernel(x, category_id, emb0_W, emb1_W, att0_Wq, att0_Wk, att0_Wv, att0_Wf,
           att1_Wq, att1_Wk, att1_Wv, att1_Wf, att2_Wq, att2_Wk, att2_Wv,
           att2_Wf, ds_w, us_W, conv_W, conv1_W, conv2_W, conv3_W, conv4_W):
    xt = jnp.transpose(x, (0, 2, 1))
    xt8 = jnp.zeros((B, N, 8), jnp.float32).at[:, :, :3].set(xt)
    x0 = _edge_conv(xt8, emb0_W, K0)
    x1 = _edge_conv(x0, emb1_W, K1)
    xf = jnp.concatenate([x0, x1], axis=-1)
    xf = _attention(xf, att0_Wq, att0_Wk, att0_Wv, att0_Wf, KA)
    xd, xyzd = _downsample(xf, xt8, ds_w, M)
    xd = _attention(xd, att1_Wq, att1_Wk, att1_Wv, att1_Wf, KA)
    xu = _upsample(xf, xd, xyzd, xt8, us_W)
    x_tmp_t = _attention(xu, att2_Wq, att2_Wk, att2_Wv, att2_Wf, KA)
    ymax, ysum = _head_pool(x_tmp_t, conv_W)
    yavg = ysum / N
    cid = _lrelu(jnp.einsum('oc,bcx->box', conv1_W, category_id))[:, :, 0]
    g = jnp.concatenate([ymax, yavg, cid], axis=1)
    out = _head_mlp(x_tmp_t, g, conv2_W, conv3_W, conv4_W)
    return jnp.transpose(out[:, :, :50], (0, 2, 1))
